# baseline (speedup 1.0000x reference)
import functools

import jax
import jax.numpy as jnp
from jax.experimental import pallas as pl


def _fused_mlp_kernel(x_ref, w0_ref, b0_ref, w1_ref, b1_ref, wout_ref,
                      bout_ref, out_ref):
    h = jnp.dot(x_ref[...], w0_ref[...], preferred_element_type=jnp.float32)
    h = h + b0_ref[...]
    h = jnp.dot(h, w1_ref[...], preferred_element_type=jnp.float32)
    h = h + b1_ref[...]
    h = jnp.dot(h, wout_ref[...], preferred_element_type=jnp.float32)
    out_ref[...] = h + bout_ref[...]


@functools.partial(jax.jit, static_argnames=("tile",))
def _run(x, W0, b0, W1, b1, W_out, b_out, tile=2048):
    batch, in_dim = x.shape
    hidden = W0.shape[1]
    out_dim = W_out.shape[1]
    n_tiles = batch // tile

    b0r = b0.reshape(1, hidden)
    b1r = b1.reshape(1, hidden)
    boutr = b_out.reshape(1, out_dim)

    full = lambda shape: pl.BlockSpec(shape, lambda i: (0, 0))
    return pl.pallas_call(
        _fused_mlp_kernel,
        grid=(n_tiles,),
        in_specs=[
            pl.BlockSpec((tile, in_dim), lambda i: (i, 0)),
            full((in_dim, hidden)),
            full((1, hidden)),
            full((hidden, hidden)),
            full((1, hidden)),
            full((hidden, out_dim)),
            full((1, out_dim)),
        ],
        out_specs=pl.BlockSpec((tile, out_dim), lambda i: (i, 0)),
        out_shape=jax.ShapeDtypeStruct((batch, out_dim), jnp.float32),
    )(x, W0, b0r, W1, b1r, W_out, boutr)


def kernel(x, W0, b0, W1, b1, W_out, b_out):
    return _run(x, W0, b0, W1, b1, W_out, b_out)


# speedup vs baseline: 1.0801x; 1.0801x over previous
import functools

import jax
import jax.numpy as jnp
from jax.experimental import pallas as pl
from jax.experimental.pallas import tpu as pltpu


def _folded_mlp_kernel(x_ref, w0_ref, b0_ref, w1_ref, b1_ref, wout_ref,
                       bout_ref, out_ref, weff_ref, beff_ref):
    @pl.when(pl.program_id(0) == 0)
    def _fold_weights():
        w01 = jnp.dot(w0_ref[...], w1_ref[...],
                      preferred_element_type=jnp.float32)
        weff_ref[...] = jnp.dot(w01, wout_ref[...],
                                preferred_element_type=jnp.float32)
        b01 = jnp.dot(b0_ref[...], w1_ref[...],
                      preferred_element_type=jnp.float32) + b1_ref[...]
        beff_ref[...] = jnp.dot(b01, wout_ref[...],
                                preferred_element_type=jnp.float32) + bout_ref[...]

    out_ref[...] = jnp.dot(x_ref[...], weff_ref[...],
                           preferred_element_type=jnp.float32) + beff_ref[...]


@functools.partial(jax.jit, static_argnames=("tile",))
def _run(x, W0, b0, W1, b1, W_out, b_out, tile=2048):
    batch, in_dim = x.shape
    hidden = W0.shape[1]
    out_dim = W_out.shape[1]
    n_tiles = batch // tile

    b0r = b0.reshape(1, hidden)
    b1r = b1.reshape(1, hidden)
    boutr = b_out.reshape(1, out_dim)

    full = lambda shape: pl.BlockSpec(shape, lambda i: (0, 0))
    return pl.pallas_call(
        _folded_mlp_kernel,
        grid=(n_tiles,),
        in_specs=[
            pl.BlockSpec((tile, in_dim), lambda i: (i, 0)),
            full((in_dim, hidden)),
            full((1, hidden)),
            full((hidden, hidden)),
            full((1, hidden)),
            full((hidden, out_dim)),
            full((1, out_dim)),
        ],
        out_specs=pl.BlockSpec((tile, out_dim), lambda i: (i, 0)),
        out_shape=jax.ShapeDtypeStruct((batch, out_dim), jnp.float32),
        scratch_shapes=[
            pltpu.VMEM((in_dim, out_dim), jnp.float32),
            pltpu.VMEM((1, out_dim), jnp.float32),
        ],
    )(x, W0, b0r, W1, b1r, W_out, boutr)


def kernel(x, W0, b0, W1, b1, W_out, b_out):
    return _run(x, W0, b0, W1, b1, W_out, b_out)


# speedup vs baseline: 1.2242x; 1.1335x over previous
import functools

import jax
import jax.numpy as jnp
from jax.experimental import pallas as pl
from jax.experimental.pallas import tpu as pltpu


def _folded_mlp_kernel(x_ref, w0_ref, b0_ref, w1_ref, b1_ref, wout_ref,
                       bout_ref, out_ref, weff_ref, beff_ref):
    @pl.when(pl.program_id(0) == 0)
    def _fold_weights():
        w01 = jnp.dot(w0_ref[...], w1_ref[...],
                      preferred_element_type=jnp.float32)
        weff_ref[...] = jnp.dot(w01, wout_ref[...],
                                preferred_element_type=jnp.float32)
        b01 = jnp.dot(b0_ref[...], w1_ref[...],
                      preferred_element_type=jnp.float32) + b1_ref[...]
        beff_ref[...] = jnp.dot(b01, wout_ref[...],
                                preferred_element_type=jnp.float32) + bout_ref[...]

    out_ref[...] = jnp.dot(x_ref[...], weff_ref[...],
                           preferred_element_type=jnp.float32) + beff_ref[...]


@functools.partial(jax.jit, static_argnames=("tile",))
def _run(x, W0, b0, W1, b1, W_out, b_out, tile=2048):
    batch, in_dim = x.shape
    hidden = W0.shape[1]
    out_dim = W_out.shape[1]
    n_tiles = batch // tile

    b0r = b0.reshape(1, hidden)
    b1r = b1.reshape(1, hidden)
    boutr = b_out.reshape(1, out_dim)

    full = lambda shape: pl.BlockSpec(shape, lambda i: (0, 0))
    return pl.pallas_call(
        _folded_mlp_kernel,
        grid=(n_tiles,),
        in_specs=[
            pl.BlockSpec((tile, in_dim), lambda i: (i, 0)),
            full((in_dim, hidden)),
            full((1, hidden)),
            full((hidden, hidden)),
            full((1, hidden)),
            full((hidden, out_dim)),
            full((1, out_dim)),
        ],
        out_specs=pl.BlockSpec((tile, out_dim), lambda i: (i, 0)),
        out_shape=jax.ShapeDtypeStruct((batch, out_dim), jnp.float32),
        scratch_shapes=[
            pltpu.VMEM((in_dim, out_dim), jnp.float32),
            pltpu.VMEM((1, out_dim), jnp.float32),
        ],
    )(x, W0, b0r, W1, b1r, W_out, boutr)


def kernel(x, W0, b0, W1, b1, W_out, b_out):
    return _run(x, W0, b0, W1, b1, W_out, b_out, tile=16384)
